# baseline (speedup 1.0000x reference)
import functools
import math

import jax
import jax.numpy as jnp
from jax import lax
from jax.experimental import pallas as pl
from jax.experimental.pallas import tpu as pltpu
from jax.experimental.pallas import tpu_sc as plsc

HID = 128
LANES = 16
VPR = HID // LANES

_info = plsc.get_sparse_core_info()
NC, NS = _info.num_cores, _info.num_subcores
NW = NC * NS

SCALE = math.sqrt(float(HID))
NSLOT = 2


def _make_kernel(b: int, s: int):
    assert s % NW == 0
    spw = s // NW
    ch = min(64, spw)
    n_sub = spw // ch

    mesh = plsc.VectorSubcoreMesh(core_axis_name="c", subcore_axis_name="s")

    @functools.partial(
        pl.kernel,
        out_type=jax.ShapeDtypeStruct((b, s, HID), jnp.float32),
        mesh=mesh,
        scratch_types=[
            pltpu.VMEM((b, spw), jnp.int32),
            [pltpu.VMEM((ch, HID), jnp.float32)] * NSLOT,
            [[pltpu.VMEM((ch, HID), jnp.float32)] * b] * NSLOT,
            [[pltpu.SemaphoreType.DMA] * b] * NSLOT,
            [[pltpu.SemaphoreType.DMA] * b] * NSLOT,
            [pltpu.SemaphoreType.DMA] * NSLOT,
            pltpu.SemaphoreType.DMA,
        ],
    )
    def body(tok_hbm, idx_hbm, pos_hbm, out_hbm, idx_v, pos_s, bufs,
             gsems, osems, psems, isem):
        wid = lax.axis_index("s") * NC + lax.axis_index("c")
        s_base = wid * spw

        icp = pltpu.async_copy(idx_hbm.at[:, pl.ds(s_base, spw)], idx_v, isem)

        scale = jnp.full((LANES,), SCALE, dtype=jnp.float32)

        def start_chunk(h):
            sl = h % NSLOT
            g = [
                pltpu.async_copy(
                    tok_hbm.at[idx_v.at[bb, pl.ds(h * ch, ch)]],
                    bufs[sl][bb], gsems[sl][bb])
                for bb in range(b)
            ]
            p = pltpu.async_copy(
                pos_hbm.at[pl.ds(s_base + h * ch, ch)], pos_s[sl], psems[sl])
            return g + [p]

        depth = min(NSLOT, n_sub)
        icp.wait()
        copies = {h: start_chunk(h) for h in range(depth)}
        out_copies = {}
        for h in range(n_sub):
            sl = h % NSLOT
            for c in copies[h]:
                c.wait()
            slot = bufs[sl]
            pos_v = pos_s[sl]

            @plsc.parallel_loop(0, ch, unroll=1)
            def row(r):
                for j in range(VPR):
                    vsl = pl.ds(j * LANES, LANES)
                    pv = pos_v[r, vsl]
                    for bb in range(b):
                        slot[bb][r, vsl] = slot[bb][r, vsl] * scale + pv

            out_copies[h] = [
                pltpu.async_copy(
                    slot[bb], out_hbm.at[bb, pl.ds(s_base + h * ch, ch)],
                    osems[sl][bb])
                for bb in range(b)
            ]
            if h + depth < n_sub:
                for c in out_copies[h + depth - NSLOT]:
                    c.wait()
                copies[h + depth] = start_chunk(h + depth)
        for h in range(max(0, n_sub - NSLOT), n_sub):
            for c in out_copies[h]:
                c.wait()

    return body


@jax.jit
def kernel(x, token_table, pos_table):
    b, s = x.shape
    out = _make_kernel(b, s)(token_table, x.astype(jnp.int32), pos_table)
    return out


# speedup vs baseline: 1.0546x; 1.0546x over previous
import functools
import math

import jax
import jax.numpy as jnp
from jax import lax
from jax.experimental import pallas as pl
from jax.experimental.pallas import tpu as pltpu
from jax.experimental.pallas import tpu_sc as plsc

HID = 128
LANES = 16
VPR = HID // LANES

_info = plsc.get_sparse_core_info()
NC, NS = _info.num_cores, _info.num_subcores
NW = NC * NS

SCALE = math.sqrt(float(HID))
NBUF = 8


def _make_kernel(b: int, s: int):
    assert s % NW == 0 and b % 2 == 0
    spw = s // NW
    ch = min(64, spw)
    cpb = spw // ch
    n_pair = (b // 2) * cpb

    mesh = plsc.VectorSubcoreMesh(core_axis_name="c", subcore_axis_name="s")

    @functools.partial(
        pl.kernel,
        out_type=jax.ShapeDtypeStruct((b, s, HID), jnp.float32),
        mesh=mesh,
        scratch_types=[
            pltpu.VMEM((b, spw), jnp.int32),
            pltpu.VMEM((spw, HID), jnp.float32),
            [pltpu.VMEM((ch, HID), jnp.float32)] * NBUF,
            [pltpu.SemaphoreType.DMA] * NBUF,
            [pltpu.SemaphoreType.DMA] * NBUF,
            pltpu.SemaphoreType.DMA,
            pltpu.SemaphoreType.DMA,
        ],
    )
    def body(tok_hbm, idx_hbm, pos_hbm, out_hbm, idx_v, pos_v, bufs,
             gsems, osems, isem, psem):
        wid = lax.axis_index("s") * NC + lax.axis_index("c")
        s_base = wid * spw

        icp = pltpu.async_copy(idx_hbm.at[:, pl.ds(s_base, spw)], idx_v, isem)
        pcp = pltpu.async_copy(pos_hbm.at[pl.ds(s_base, spw)], pos_v, psem)

        scale = jnp.full((LANES,), SCALE, dtype=jnp.float32)

        def slots(k):
            return (2 * k) % NBUF, (2 * k + 1) % NBUF

        def start_gathers(k):
            bp, h = k // cpb, k % cpb
            sl2 = slots(k)
            return [
                pltpu.async_copy(
                    tok_hbm.at[idx_v.at[2 * bp + i, pl.ds(h * ch, ch)]],
                    bufs[sl2[i]], gsems[sl2[i]])
                for i in range(2)
            ]

        depth = min(NBUF // 2 - 1, n_pair)
        icp.wait()
        copies = {k: start_gathers(k) for k in range(depth)}
        out_copies = {}
        pcp.wait()
        for k in range(n_pair):
            if k + depth < n_pair:
                prev = k + depth - NBUF // 2
                if prev >= 0:
                    for c in out_copies[prev]:
                        c.wait()
                copies[k + depth] = start_gathers(k + depth)
            for c in copies[k]:
                c.wait()
            sa, sb = slots(k)
            buf_a, buf_b = bufs[sa], bufs[sb]
            bp, h = k // cpb, k % cpb
            pbase = h * ch

            @plsc.parallel_loop(0, ch, unroll=1)
            def row(r):
                for j in range(VPR):
                    sl = pl.ds(j * LANES, LANES)
                    pv = pos_v[pbase + r, sl]
                    buf_a[r, sl] = buf_a[r, sl] * scale + pv
                    buf_b[r, sl] = buf_b[r, sl] * scale + pv

            out_copies[k] = [
                pltpu.async_copy(
                    bufs[(sa, sb)[i]],
                    out_hbm.at[2 * bp + i, pl.ds(s_base + h * ch, ch)],
                    osems[(sa, sb)[i]])
                for i in range(2)
            ]
        for k in range(max(0, n_pair - NBUF // 2), n_pair):
            for c in out_copies[k]:
                c.wait()

    return body


@jax.jit
def kernel(x, token_table, pos_table):
    b, s = x.shape
    out = _make_kernel(b, s)(token_table, x.astype(jnp.int32), pos_table)
    return out
